# baseline (speedup 1.0000x reference)
import functools

import jax
import jax.numpy as jnp
import numpy as np
from jax.experimental import pallas as pl
from jax.experimental.pallas import tpu as pltpu

_BN_EPS = 1e-5
_POOLK = (4, 8, 16, 32)


def _pool1d(n, k):
    out = n // k
    m = np.zeros((out, n), np.float32)
    for i in range(out):
        m[i, i * k:(i + 1) * k] = 1.0 / k
    return m


def _up1d(n_out, n_in):
    m = np.zeros((n_out, n_in), np.float32)
    for i in range(n_out):
        p = i * (n_in - 1) / (n_out - 1) if n_out > 1 else 0.0
        lo = int(np.floor(p))
        hi = min(lo + 1, n_in - 1)
        f = p - lo
        m[i, lo] += 1.0 - f
        m[i, hi] += f
    return m


def _build_operators(H, W):
    HW = H * W
    hw_list = [(H // k) * (W // k) for k in _POOLK]
    p_list = [max(8, -(-hw // 8) * 8) for hw in hw_list]
    p_bases = np.cumsum([0] + p_list)[:-1].tolist()
    psum = sum(p_list)

    a_stack = np.zeros((psum, HW), np.float32)
    u_list = []
    for i, k in enumerate(_POOLK):
        a_stack[p_bases[i]:p_bases[i] + hw_list[i]] = np.kron(
            _pool1d(H, k), _pool1d(W, k))
        u_list.append(np.kron(_up1d(H, H // k), _up1d(W, W // k)))

    r_bases = np.cumsum([0] + [9 * p for p in p_list])[:-1].tolist()
    l_op = np.zeros((HW, 9 * psum), np.float32)
    yy, xx = np.meshgrid(np.arange(H), np.arange(W), indexing="ij")
    for t in range(9):
        dy, dx = t // 3 - 1, t % 3 - 1
        sy, sx = (yy + dy).ravel(), (xx + dx).ravel()
        ok = (sy >= 0) & (sy < H) & (sx >= 0) & (sx < W)
        src = np.where(ok, sy * W + sx, 0)
        for i, (u, hw) in enumerate(zip(u_list, hw_list)):
            col0 = r_bases[i] + t * p_list[i]
            l_op[:, col0:col0 + hw] = np.where(ok[:, None], u[src], 0.0)
    return a_stack, l_op, p_list, p_bases, r_bases


def _spp_fused_kernel(x_ref, a_ref, l_ref, w1_ref, s1_ref, b1_ref, wt_ref,
                      wx3_ref, so_ref, bo_ref, o_ref, xq_ref, r_ref, *, H, W,
                      p_list, p_bases, r_bases):
    C = o_ref.shape[-2]
    HW = H * W

    x = x_ref[0]
    dn11 = (((1,), (1,)), ((), ()))

    pooled = jnp.dot(a_ref[...], x, preferred_element_type=jnp.float32)
    for i in range(4):
        pb, pn, rb = p_bases[i], p_list[i], r_bases[i]
        g = jnp.maximum(
            jax.lax.dot_general(pooled[pb:pb + pn].astype(jnp.bfloat16),
                                w1_ref[i], dn11,
                                preferred_element_type=jnp.float32)
            * s1_ref[i] + b1_ref[i], 0.0)
        gb = g.astype(jnp.bfloat16)
        for t in range(9):
            res = jax.lax.dot_general(
                gb, wt_ref[t, :, i * C:(i + 1) * C], dn11,
                preferred_element_type=jnp.float32)
            r_ref[rb + t * pn:rb + (t + 1) * pn, :] = res.astype(jnp.bfloat16)

    xq_ref[...] = jnp.zeros(xq_ref.shape, xq_ref.dtype)
    xq_ref[W:W + HW, C:2 * C] = x
    xq_ref[W + 1:W + HW, 0:C] = x[:HW - 1, :]
    xq_ref[W:W + HW - 1, 2 * C:3 * C] = x[1:, :]
    zrow = jnp.zeros((1, C), jnp.bfloat16)
    for q in range(2, H + 1):
        xq_ref[q * W:q * W + 1, 0:C] = zrow
    for q in range(1, H):
        xq_ref[(q + 1) * W - 1:(q + 1) * W, 2 * C:3 * C] = zrow

    dn01 = (((0,), (1,)), ((), ()))
    acc = jax.lax.dot_general(r_ref[...], l_ref[...], dn01,
                              preferred_element_type=jnp.float32)
    for di in range(3):
        acc = acc + jax.lax.dot_general(
            wx3_ref[di], xq_ref[pl.ds(di * W, HW), :], dn01,
            preferred_element_type=jnp.float32)
    o_ref[0] = jnp.maximum(acc * so_ref[...] + bo_ref[...], 0.0)


def kernel(x, branch0_w, branch0_b, branch0_gamma, branch0_beta, branch0_mean,
           branch0_var, branch1_w, branch1_b, branch1_gamma, branch1_beta,
           branch1_mean, branch1_var, branch2_w, branch2_b, branch2_gamma,
           branch2_beta, branch2_mean, branch2_var, branch3_w, branch3_b,
           branch3_gamma, branch3_beta, branch3_mean, branch3_var,
           out_w, out_b, out_gamma, out_beta, out_mean, out_var):
    B, C, H, W = x.shape
    HW = H * W

    a_np, l_np, p_list, p_bases, r_bases = _build_operators(H, W)
    a_stack = jnp.asarray(a_np, jnp.bfloat16)
    l_op = jnp.asarray(l_np, jnp.bfloat16)
    psum = sum(p_list)

    s1_l, b1_l = [], []
    for b, gamma, beta, mean, var in (
            (branch0_b, branch0_gamma, branch0_beta, branch0_mean, branch0_var),
            (branch1_b, branch1_gamma, branch1_beta, branch1_mean, branch1_var),
            (branch2_b, branch2_gamma, branch2_beta, branch2_mean, branch2_var),
            (branch3_b, branch3_gamma, branch3_beta, branch3_mean, branch3_var)):
        s = gamma * jax.lax.rsqrt(var + _BN_EPS)
        s1_l.append(s)
        b1_l.append(s * (b - mean) + beta)
    w1 = jnp.stack([branch0_w, branch1_w, branch2_w,
                    branch3_w]).astype(jnp.bfloat16)
    s1 = jnp.stack(s1_l).reshape(4, 1, C)
    b1 = jnp.stack(b1_l).reshape(4, 1, C)

    w_bf = out_w.astype(jnp.bfloat16)
    wt = jnp.transpose(w_bf[:, :4 * C], (2, 3, 0, 1)).reshape(9, C, 4 * C)
    wx3 = jnp.transpose(w_bf[:, 4 * C:].reshape(C, C, 3, 3),
                        (2, 3, 1, 0)).reshape(3, 3 * C, C)
    so = (out_gamma * jax.lax.rsqrt(out_var + _BN_EPS)).reshape(C, 1)
    bo = (so[:, 0] * (out_b - out_mean) + out_beta).reshape(C, 1)

    x_cl = jnp.transpose(x.astype(jnp.bfloat16).reshape(B, C, HW),
                         (0, 2, 1))

    flops = B * 2 * (psum * HW * C + psum * C * C + 9 * psum * C * C
                     + HW * 9 * psum * C + 3 * HW * 3 * C * C)
    bytes_accessed = 4 * (B * HW * C * 2) + 2 * (HW * 9 * psum
                                                 + 4 * C * 9 * C + 9 * C * C)
    out_cf = pl.pallas_call(
        functools.partial(_spp_fused_kernel, H=H, W=W, p_list=p_list,
                          p_bases=p_bases, r_bases=r_bases),
        out_shape=jax.ShapeDtypeStruct((B, C, HW), jnp.float32),
        grid=(B,),
        in_specs=[
            pl.BlockSpec((1, HW, C), lambda b: (b, 0, 0)),
            pl.BlockSpec((psum, HW), lambda b: (0, 0)),
            pl.BlockSpec((HW, 9 * psum), lambda b: (0, 0)),
            pl.BlockSpec((4, C, C), lambda b: (0, 0, 0)),
            pl.BlockSpec((4, 1, C), lambda b: (0, 0, 0)),
            pl.BlockSpec((4, 1, C), lambda b: (0, 0, 0)),
            pl.BlockSpec((9, C, 4 * C), lambda b: (0, 0, 0)),
            pl.BlockSpec((3, 3 * C, C), lambda b: (0, 0, 0)),
            pl.BlockSpec((C, 1), lambda b: (0, 0)),
            pl.BlockSpec((C, 1), lambda b: (0, 0)),
        ],
        out_specs=pl.BlockSpec((1, C, HW), lambda b: (b, 0, 0)),
        scratch_shapes=[pltpu.VMEM(((H + 2) * W, 3 * C), jnp.bfloat16),
                        pltpu.VMEM((9 * psum, C), jnp.bfloat16)],
        compiler_params=pltpu.CompilerParams(
            dimension_semantics=("parallel",),
            vmem_limit_bytes=48 * 1024 * 1024,
        ),
        cost_estimate=pl.CostEstimate(flops=flops, transcendentals=0,
                                      bytes_accessed=bytes_accessed),
    )(x_cl, a_stack, l_op, w1, s1, b1, wt, wx3, so, bo)

    return out_cf.reshape(B, C, H, W)


# speedup vs baseline: 1.0735x; 1.0735x over previous
import functools

import jax
import jax.numpy as jnp
import numpy as np
from jax.experimental import pallas as pl
from jax.experimental.pallas import tpu as pltpu

_BN_EPS = 1e-5
_POOLK = (4, 8, 16, 32)


def _pool1d(n, k):
    out = n // k
    m = np.zeros((out, n), np.float32)
    for i in range(out):
        m[i, i * k:(i + 1) * k] = 1.0 / k
    return m


def _up1d(n_out, n_in):
    m = np.zeros((n_out, n_in), np.float32)
    for i in range(n_out):
        p = i * (n_in - 1) / (n_out - 1) if n_out > 1 else 0.0
        lo = int(np.floor(p))
        hi = min(lo + 1, n_in - 1)
        f = p - lo
        m[i, lo] += 1.0 - f
        m[i, hi] += f
    return m


def _build_operators(H, W):
    HW = H * W
    hw_list = [(H // k) * (W // k) for k in _POOLK]
    p_list = [max(8, -(-hw // 8) * 8) for hw in hw_list]
    p_bases = np.cumsum([0] + p_list)[:-1].tolist()
    psum = sum(p_list)

    a_stack = np.zeros((psum, HW), np.float32)
    u_list = []
    for i, k in enumerate(_POOLK):
        a_stack[p_bases[i]:p_bases[i] + hw_list[i]] = np.kron(
            _pool1d(H, k), _pool1d(W, k))
        u_list.append(np.kron(_up1d(H, H // k), _up1d(W, W // k)))

    r_bases = np.cumsum([0] + [9 * p for p in p_list])[:-1].tolist()
    l_op = np.zeros((HW, 9 * psum), np.float32)
    yy, xx = np.meshgrid(np.arange(H), np.arange(W), indexing="ij")
    for t in range(9):
        dy, dx = t // 3 - 1, t % 3 - 1
        sy, sx = (yy + dy).ravel(), (xx + dx).ravel()
        ok = (sy >= 0) & (sy < H) & (sx >= 0) & (sx < W)
        src = np.where(ok, sy * W + sx, 0)
        for i, (u, hw) in enumerate(zip(u_list, hw_list)):
            col0 = r_bases[i] + t * p_list[i]
            l_op[:, col0:col0 + hw] = np.where(ok[:, None], u[src], 0.0)
    return a_stack, l_op, p_list, p_bases, r_bases


def _spp_fused_kernel(x_ref, a_ref, l_ref, w1_ref, s1_ref, b1_ref, wt_ref,
                      so_ref, bo_ref, o_ref, xq_ref, r_ref, *, H, W, p_list,
                      p_bases, r_bases):
    C = o_ref.shape[-2]
    HW = H * W

    x = x_ref[0]
    dn11 = (((1,), (1,)), ((), ()))

    pooled = jnp.dot(a_ref[...], x, preferred_element_type=jnp.float32)
    for i in range(4):
        pb, pn, rb = p_bases[i], p_list[i], r_bases[i]
        g = jnp.maximum(
            jax.lax.dot_general(pooled[pb:pb + pn].astype(jnp.bfloat16),
                                w1_ref[i], dn11,
                                preferred_element_type=jnp.float32)
            * s1_ref[i] + b1_ref[i], 0.0)
        gb = g.astype(jnp.bfloat16)
        for t in range(9):
            res = jax.lax.dot_general(
                gb, wt_ref[t, :, i * C:(i + 1) * C], dn11,
                preferred_element_type=jnp.float32)
            r_ref[rb + t * pn:rb + (t + 1) * pn, :] = res.astype(jnp.bfloat16)

    xq_ref[...] = jnp.zeros(xq_ref.shape, xq_ref.dtype)
    xq_ref[W:W + HW, C:2 * C] = x
    xq_ref[W + 1:W + HW, 0:C] = x[:HW - 1, :]
    xq_ref[W:W + HW - 1, 2 * C:3 * C] = x[1:, :]
    zrow = jnp.zeros((1, C), jnp.bfloat16)
    for q in range(2, H + 1):
        xq_ref[q * W:q * W + 1, 0:C] = zrow
    for q in range(1, H):
        xq_ref[(q + 1) * W - 1:(q + 1) * W, 2 * C:3 * C] = zrow

    dn01 = (((0,), (1,)), ((), ()))
    acc = jax.lax.dot_general(r_ref[...], l_ref[...], dn01,
                              preferred_element_type=jnp.float32)
    for t in range(9):
        di, dj = t // 3, t % 3
        acc = acc + jax.lax.dot_general(
            wt_ref[t, :, 4 * C:], xq_ref[pl.ds(di * W, HW),
                                         dj * C:(dj + 1) * C], dn11,
            preferred_element_type=jnp.float32)
    o_ref[0] = jnp.maximum(acc * so_ref[...] + bo_ref[...], 0.0)


def kernel(x, branch0_w, branch0_b, branch0_gamma, branch0_beta, branch0_mean,
           branch0_var, branch1_w, branch1_b, branch1_gamma, branch1_beta,
           branch1_mean, branch1_var, branch2_w, branch2_b, branch2_gamma,
           branch2_beta, branch2_mean, branch2_var, branch3_w, branch3_b,
           branch3_gamma, branch3_beta, branch3_mean, branch3_var,
           out_w, out_b, out_gamma, out_beta, out_mean, out_var):
    B, C, H, W = x.shape
    HW = H * W

    a_np, l_np, p_list, p_bases, r_bases = _build_operators(H, W)
    a_stack = jnp.asarray(a_np, jnp.bfloat16)
    l_op = jnp.asarray(l_np, jnp.bfloat16)
    psum = sum(p_list)

    s1_l, b1_l = [], []
    for b, gamma, beta, mean, var in (
            (branch0_b, branch0_gamma, branch0_beta, branch0_mean, branch0_var),
            (branch1_b, branch1_gamma, branch1_beta, branch1_mean, branch1_var),
            (branch2_b, branch2_gamma, branch2_beta, branch2_mean, branch2_var),
            (branch3_b, branch3_gamma, branch3_beta, branch3_mean, branch3_var)):
        s = gamma * jax.lax.rsqrt(var + _BN_EPS)
        s1_l.append(s)
        b1_l.append(s * (b - mean) + beta)
    w1 = jnp.stack([branch0_w, branch1_w, branch2_w,
                    branch3_w]).astype(jnp.bfloat16)
    s1 = jnp.stack(s1_l).reshape(4, 1, C)
    b1 = jnp.stack(b1_l).reshape(4, 1, C)

    wt = jnp.transpose(out_w.astype(jnp.bfloat16),
                       (2, 3, 0, 1)).reshape(9, C, 5 * C)
    so = (out_gamma * jax.lax.rsqrt(out_var + _BN_EPS)).reshape(C, 1)
    bo = (so[:, 0] * (out_b - out_mean) + out_beta).reshape(C, 1)

    x_cl = jnp.transpose(x.astype(jnp.bfloat16).reshape(B, C, HW),
                         (0, 2, 1))

    flops = B * 2 * (psum * HW * C + psum * C * C + 9 * psum * C * C
                     + HW * 9 * psum * C + 3 * HW * 3 * C * C)
    bytes_accessed = 4 * (B * HW * C * 2) + 2 * (HW * 9 * psum
                                                 + 4 * C * 9 * C + 9 * C * C)
    out_cf = pl.pallas_call(
        functools.partial(_spp_fused_kernel, H=H, W=W, p_list=p_list,
                          p_bases=p_bases, r_bases=r_bases),
        out_shape=jax.ShapeDtypeStruct((B, C, HW), jnp.float32),
        grid=(B,),
        in_specs=[
            pl.BlockSpec((1, HW, C), lambda b: (b, 0, 0)),
            pl.BlockSpec((psum, HW), lambda b: (0, 0)),
            pl.BlockSpec((HW, 9 * psum), lambda b: (0, 0)),
            pl.BlockSpec((4, C, C), lambda b: (0, 0, 0)),
            pl.BlockSpec((4, 1, C), lambda b: (0, 0, 0)),
            pl.BlockSpec((4, 1, C), lambda b: (0, 0, 0)),
            pl.BlockSpec((9, C, 5 * C), lambda b: (0, 0, 0)),
            pl.BlockSpec((C, 1), lambda b: (0, 0)),
            pl.BlockSpec((C, 1), lambda b: (0, 0)),
        ],
        out_specs=pl.BlockSpec((1, C, HW), lambda b: (b, 0, 0)),
        scratch_shapes=[pltpu.VMEM(((H + 2) * W, 3 * C), jnp.bfloat16),
                        pltpu.VMEM((9 * psum, C), jnp.bfloat16)],
        compiler_params=pltpu.CompilerParams(
            dimension_semantics=("parallel",),
            vmem_limit_bytes=48 * 1024 * 1024,
        ),
        cost_estimate=pl.CostEstimate(flops=flops, transcendentals=0,
                                      bytes_accessed=bytes_accessed),
    )(x_cl, a_stack, l_op, w1, s1, b1, wt, so, bo)

    return out_cf.reshape(B, C, H, W)


# speedup vs baseline: 1.2614x; 1.1750x over previous
import functools

import jax
import jax.numpy as jnp
import numpy as np
from jax.experimental import pallas as pl
from jax.experimental.pallas import tpu as pltpu

_BN_EPS = 1e-5
_POOLK = (4, 8, 16, 32)


def _pool1d(n, k):
    out = n // k
    m = np.zeros((out, n), np.float32)
    for i in range(out):
        m[i, i * k:(i + 1) * k] = 1.0 / k
    return m


def _up1d(n_out, n_in):
    m = np.zeros((n_out, n_in), np.float32)
    for i in range(n_out):
        p = i * (n_in - 1) / (n_out - 1) if n_out > 1 else 0.0
        lo = int(np.floor(p))
        hi = min(lo + 1, n_in - 1)
        f = p - lo
        m[i, lo] += 1.0 - f
        m[i, hi] += f
    return m


def _build_operators(H, W):
    HW = H * W
    hw_list = [(H // k) * (W // k) for k in _POOLK]
    p_list = [max(8, -(-hw // 8) * 8) for hw in hw_list]
    p_bases = np.cumsum([0] + p_list)[:-1].tolist()
    psum = sum(p_list)

    a_stack = np.zeros((psum, HW), np.float32)
    u_list = []
    for i, k in enumerate(_POOLK):
        a_stack[p_bases[i]:p_bases[i] + hw_list[i]] = np.kron(
            _pool1d(H, k), _pool1d(W, k))
        u_list.append(np.kron(_up1d(H, H // k), _up1d(W, W // k)))

    r_bases = np.cumsum([0] + [9 * p for p in p_list])[:-1].tolist()
    l_op = np.zeros((HW, 9 * psum), np.float32)
    yy, xx = np.meshgrid(np.arange(H), np.arange(W), indexing="ij")
    for t in range(9):
        dy, dx = t // 3 - 1, t % 3 - 1
        sy, sx = (yy + dy).ravel(), (xx + dx).ravel()
        ok = (sy >= 0) & (sy < H) & (sx >= 0) & (sx < W)
        src = np.where(ok, sy * W + sx, 0)
        for i, (u, hw) in enumerate(zip(u_list, hw_list)):
            col0 = r_bases[i] + t * p_list[i]
            l_op[:, col0:col0 + hw] = np.where(ok[:, None], u[src], 0.0)
    return a_stack, l_op, p_list, p_bases, r_bases


def _spp_fused_kernel(x_ref, a_ref, l_ref, w1_ref, s1_ref, b1_ref, wt_ref,
                      so_ref, bo_ref, o_ref, xq_ref, r_ref, *, H, W, p_list,
                      p_bases, r_bases):
    C = o_ref.shape[-2]
    HW = H * W
    NI = o_ref.shape[0]
    KL = r_ref.shape[0] // NI
    QR = xq_ref.shape[0] // NI
    dn11 = (((1,), (1,)), ((), ()))

    pooled = [jnp.dot(a_ref[...], x_ref[n],
                      preferred_element_type=jnp.float32) for n in range(NI)]
    for i in range(4):
        pb, pn, rb = p_bases[i], p_list[i], r_bases[i]
        p2 = jnp.concatenate([p[pb:pb + pn] for p in pooled], axis=0)
        g = jnp.maximum(
            jax.lax.dot_general(p2.astype(jnp.bfloat16), w1_ref[i], dn11,
                                preferred_element_type=jnp.float32)
            * s1_ref[i] + b1_ref[i], 0.0)
        gb = g.astype(jnp.bfloat16)
        for t in range(9):
            res = jax.lax.dot_general(
                gb, wt_ref[t, :, i * C:(i + 1) * C], dn11,
                preferred_element_type=jnp.float32).astype(jnp.bfloat16)
            for n in range(NI):
                r_ref[n * KL + rb + t * pn:n * KL + rb + (t + 1) * pn, :] = \
                    res[n * pn:(n + 1) * pn]

    xq_ref[...] = jnp.zeros(xq_ref.shape, xq_ref.dtype)
    zrow = jnp.zeros((1, C), jnp.bfloat16)
    for n in range(NI):
        x = x_ref[n]
        q0 = n * QR
        xq_ref[q0 + W:q0 + W + HW, C:2 * C] = x
        xq_ref[q0 + W + 1:q0 + W + HW, 0:C] = x[:HW - 1, :]
        xq_ref[q0 + W:q0 + W + HW - 1, 2 * C:3 * C] = x[1:, :]
        for q in range(2, H + 1):
            xq_ref[q0 + q * W:q0 + q * W + 1, 0:C] = zrow
        for q in range(1, H):
            xq_ref[q0 + (q + 1) * W - 1:q0 + (q + 1) * W, 2 * C:3 * C] = zrow

    dn01 = (((0,), (1,)), ((), ()))
    accs = [jax.lax.dot_general(r_ref[pl.ds(n * KL, KL), :], l_ref[...], dn01,
                                preferred_element_type=jnp.float32)
            for n in range(NI)]
    for t in range(9):
        di, dj = t // 3, t % 3
        for n in range(NI):
            accs[n] = accs[n] + jax.lax.dot_general(
                wt_ref[t, :, 4 * C:],
                xq_ref[pl.ds(n * QR + di * W, HW), dj * C:(dj + 1) * C], dn11,
                preferred_element_type=jnp.float32)
    for n in range(NI):
        o_ref[n] = jnp.maximum(accs[n] * so_ref[...] + bo_ref[...], 0.0)


def kernel(x, branch0_w, branch0_b, branch0_gamma, branch0_beta, branch0_mean,
           branch0_var, branch1_w, branch1_b, branch1_gamma, branch1_beta,
           branch1_mean, branch1_var, branch2_w, branch2_b, branch2_gamma,
           branch2_beta, branch2_mean, branch2_var, branch3_w, branch3_b,
           branch3_gamma, branch3_beta, branch3_mean, branch3_var,
           out_w, out_b, out_gamma, out_beta, out_mean, out_var):
    B, C, H, W = x.shape
    HW = H * W

    a_np, l_np, p_list, p_bases, r_bases = _build_operators(H, W)
    a_stack = jnp.asarray(a_np, jnp.bfloat16)
    l_op = jnp.asarray(l_np, jnp.bfloat16)
    psum = sum(p_list)

    s1_l, b1_l = [], []
    for b, gamma, beta, mean, var in (
            (branch0_b, branch0_gamma, branch0_beta, branch0_mean, branch0_var),
            (branch1_b, branch1_gamma, branch1_beta, branch1_mean, branch1_var),
            (branch2_b, branch2_gamma, branch2_beta, branch2_mean, branch2_var),
            (branch3_b, branch3_gamma, branch3_beta, branch3_mean, branch3_var)):
        s = gamma * jax.lax.rsqrt(var + _BN_EPS)
        s1_l.append(s)
        b1_l.append(s * (b - mean) + beta)
    w1 = jnp.stack([branch0_w, branch1_w, branch2_w,
                    branch3_w]).astype(jnp.bfloat16)
    s1 = jnp.stack(s1_l).reshape(4, 1, C)
    b1 = jnp.stack(b1_l).reshape(4, 1, C)

    wt = jnp.transpose(out_w.astype(jnp.bfloat16),
                       (2, 3, 0, 1)).reshape(9, C, 5 * C)
    so = (out_gamma * jax.lax.rsqrt(out_var + _BN_EPS)).reshape(C, 1)
    bo = (so[:, 0] * (out_b - out_mean) + out_beta).reshape(C, 1)

    x_cl = jnp.transpose(x.astype(jnp.bfloat16).reshape(B, C, HW),
                         (0, 2, 1))

    flops = B * 2 * (psum * HW * C + psum * C * C + 9 * psum * C * C
                     + HW * 9 * psum * C + 3 * HW * 3 * C * C)
    bytes_accessed = 4 * (B * HW * C * 2) + 2 * (HW * 9 * psum
                                                 + 4 * C * 9 * C + 9 * C * C)
    ni = 2 if B % 2 == 0 else 1
    out_cf = pl.pallas_call(
        functools.partial(_spp_fused_kernel, H=H, W=W, p_list=p_list,
                          p_bases=p_bases, r_bases=r_bases),
        out_shape=jax.ShapeDtypeStruct((B, C, HW), jnp.float32),
        grid=(B // ni,),
        in_specs=[
            pl.BlockSpec((ni, HW, C), lambda b: (b, 0, 0)),
            pl.BlockSpec((psum, HW), lambda b: (0, 0)),
            pl.BlockSpec((HW, 9 * psum), lambda b: (0, 0)),
            pl.BlockSpec((4, C, C), lambda b: (0, 0, 0)),
            pl.BlockSpec((4, 1, C), lambda b: (0, 0, 0)),
            pl.BlockSpec((4, 1, C), lambda b: (0, 0, 0)),
            pl.BlockSpec((9, C, 5 * C), lambda b: (0, 0, 0)),
            pl.BlockSpec((C, 1), lambda b: (0, 0)),
            pl.BlockSpec((C, 1), lambda b: (0, 0)),
        ],
        out_specs=pl.BlockSpec((ni, C, HW), lambda b: (b, 0, 0)),
        scratch_shapes=[pltpu.VMEM((ni * (H + 2) * W, 3 * C), jnp.bfloat16),
                        pltpu.VMEM((ni * 9 * psum, C), jnp.bfloat16)],
        compiler_params=pltpu.CompilerParams(
            dimension_semantics=("parallel",),
            vmem_limit_bytes=48 * 1024 * 1024,
        ),
        cost_estimate=pl.CostEstimate(flops=flops, transcendentals=0,
                                      bytes_accessed=bytes_accessed),
    )(x_cl, a_stack, l_op, w1, s1, b1, wt, so, bo)

    return out_cf.reshape(B, C, H, W)


# speedup vs baseline: 1.3445x; 1.0659x over previous
import functools

import jax
import jax.numpy as jnp
import numpy as np
from jax.experimental import pallas as pl
from jax.experimental.pallas import tpu as pltpu

_BN_EPS = 1e-5
_POOLK = (4, 8, 16, 32)


def _pool1d(n, k):
    out = n // k
    m = np.zeros((out, n), np.float32)
    for i in range(out):
        m[i, i * k:(i + 1) * k] = 1.0 / k
    return m


def _up1d(n_out, n_in):
    m = np.zeros((n_out, n_in), np.float32)
    for i in range(n_out):
        p = i * (n_in - 1) / (n_out - 1) if n_out > 1 else 0.0
        lo = int(np.floor(p))
        hi = min(lo + 1, n_in - 1)
        f = p - lo
        m[i, lo] += 1.0 - f
        m[i, hi] += f
    return m


def _build_operators(H, W):
    HW = H * W
    hw_list = [(H // k) * (W // k) for k in _POOLK]
    p_list = [max(8, -(-hw // 8) * 8) for hw in hw_list]
    p_bases = np.cumsum([0] + p_list)[:-1].tolist()
    psum = sum(p_list)

    a_stack = np.zeros((psum, HW), np.float32)
    u_list = []
    for i, k in enumerate(_POOLK):
        a_stack[p_bases[i]:p_bases[i] + hw_list[i]] = np.kron(
            _pool1d(H, k), _pool1d(W, k))
        u_list.append(np.kron(_up1d(H, H // k), _up1d(W, W // k)))

    r_bases = np.cumsum([0] + [9 * p for p in p_list])[:-1].tolist()
    l_op = np.zeros((HW, 9 * psum), np.float32)
    yy, xx = np.meshgrid(np.arange(H), np.arange(W), indexing="ij")
    for t in range(9):
        dy, dx = t // 3 - 1, t % 3 - 1
        sy, sx = (yy + dy).ravel(), (xx + dx).ravel()
        ok = (sy >= 0) & (sy < H) & (sx >= 0) & (sx < W)
        src = np.where(ok, sy * W + sx, 0)
        for i, (u, hw) in enumerate(zip(u_list, hw_list)):
            col0 = r_bases[i] + t * p_list[i]
            l_op[:, col0:col0 + hw] = np.where(ok[:, None], u[src], 0.0)
    return a_stack, l_op, p_list, p_bases, r_bases


def _spp_fused_kernel(x_ref, a_ref, l_ref, w1_ref, s1_ref, b1_ref, wt_ref,
                      so_ref, bo_ref, o_ref, xq_ref, r_ref, *, H, W, p_list,
                      p_bases, r_bases):
    C = o_ref.shape[-2]
    HW = H * W
    NI = o_ref.shape[0]
    KL = r_ref.shape[0] // NI
    QR = xq_ref.shape[0] // NI
    dn11 = (((1,), (1,)), ((), ()))

    pooled = [jnp.dot(a_ref[...], x_ref[n],
                      preferred_element_type=jnp.float32) for n in range(NI)]
    for i in range(4):
        pb, pn, rb = p_bases[i], p_list[i], r_bases[i]
        p2 = jnp.concatenate([p[pb:pb + pn] for p in pooled], axis=0)
        g = jnp.maximum(
            jax.lax.dot_general(p2.astype(jnp.bfloat16), w1_ref[i], dn11,
                                preferred_element_type=jnp.float32)
            * s1_ref[i] + b1_ref[i], 0.0)
        gb = g.astype(jnp.bfloat16)
        for t in range(9):
            res = jax.lax.dot_general(
                gb, wt_ref[t, :, i * C:(i + 1) * C], dn11,
                preferred_element_type=jnp.float32).astype(jnp.bfloat16)
            for n in range(NI):
                r_ref[n * KL + rb + t * pn:n * KL + rb + (t + 1) * pn, :] = \
                    res[n * pn:(n + 1) * pn]

    xq_ref[...] = jnp.zeros(xq_ref.shape, xq_ref.dtype)
    zrow = jnp.zeros((1, C), jnp.bfloat16)
    for n in range(NI):
        x = x_ref[n]
        q0 = n * QR
        xq_ref[q0 + W:q0 + W + HW, C:2 * C] = x
        xq_ref[q0 + W + 1:q0 + W + HW, 0:C] = x[:HW - 1, :]
        xq_ref[q0 + W:q0 + W + HW - 1, 2 * C:3 * C] = x[1:, :]
        for q in range(2, H + 1):
            xq_ref[q0 + q * W:q0 + q * W + 1, 0:C] = zrow
        for q in range(1, H):
            xq_ref[q0 + (q + 1) * W - 1:q0 + (q + 1) * W, 2 * C:3 * C] = zrow

    dn01 = (((0,), (1,)), ((), ()))
    accs = [jax.lax.dot_general(r_ref[pl.ds(n * KL, KL), :], l_ref[...], dn01,
                                preferred_element_type=jnp.float32)
            for n in range(NI)]
    for t in range(9):
        di, dj = t // 3, t % 3
        for n in range(NI):
            accs[n] = accs[n] + jax.lax.dot_general(
                wt_ref[t, :, 4 * C:],
                xq_ref[pl.ds(n * QR + di * W, HW), dj * C:(dj + 1) * C], dn11,
                preferred_element_type=jnp.float32)
    for n in range(NI):
        o_ref[n] = jnp.maximum(accs[n] * so_ref[...] + bo_ref[...], 0.0)


def kernel(x, branch0_w, branch0_b, branch0_gamma, branch0_beta, branch0_mean,
           branch0_var, branch1_w, branch1_b, branch1_gamma, branch1_beta,
           branch1_mean, branch1_var, branch2_w, branch2_b, branch2_gamma,
           branch2_beta, branch2_mean, branch2_var, branch3_w, branch3_b,
           branch3_gamma, branch3_beta, branch3_mean, branch3_var,
           out_w, out_b, out_gamma, out_beta, out_mean, out_var):
    B, C, H, W = x.shape
    HW = H * W

    a_np, l_np, p_list, p_bases, r_bases = _build_operators(H, W)
    a_stack = jnp.asarray(a_np, jnp.bfloat16)
    l_op = jnp.asarray(l_np, jnp.bfloat16)
    psum = sum(p_list)

    s1_l, b1_l = [], []
    for b, gamma, beta, mean, var in (
            (branch0_b, branch0_gamma, branch0_beta, branch0_mean, branch0_var),
            (branch1_b, branch1_gamma, branch1_beta, branch1_mean, branch1_var),
            (branch2_b, branch2_gamma, branch2_beta, branch2_mean, branch2_var),
            (branch3_b, branch3_gamma, branch3_beta, branch3_mean, branch3_var)):
        s = gamma * jax.lax.rsqrt(var + _BN_EPS)
        s1_l.append(s)
        b1_l.append(s * (b - mean) + beta)
    w1 = jnp.stack([branch0_w, branch1_w, branch2_w,
                    branch3_w]).astype(jnp.bfloat16)
    s1 = jnp.stack(s1_l).reshape(4, 1, C)
    b1 = jnp.stack(b1_l).reshape(4, 1, C)

    wt = jnp.transpose(out_w.astype(jnp.bfloat16),
                       (2, 3, 0, 1)).reshape(9, C, 5 * C)
    so = (out_gamma * jax.lax.rsqrt(out_var + _BN_EPS)).reshape(C, 1)
    bo = (so[:, 0] * (out_b - out_mean) + out_beta).reshape(C, 1)

    x_cl = jnp.transpose(x.astype(jnp.bfloat16).reshape(B, C, HW),
                         (0, 2, 1))

    flops = B * 2 * (psum * HW * C + psum * C * C + 9 * psum * C * C
                     + HW * 9 * psum * C + 3 * HW * 3 * C * C)
    bytes_accessed = 4 * (B * HW * C * 2) + 2 * (HW * 9 * psum
                                                 + 4 * C * 9 * C + 9 * C * C)
    ni = 4 if B % 4 == 0 else (2 if B % 2 == 0 else 1)
    out_cf = pl.pallas_call(
        functools.partial(_spp_fused_kernel, H=H, W=W, p_list=p_list,
                          p_bases=p_bases, r_bases=r_bases),
        out_shape=jax.ShapeDtypeStruct((B, C, HW), jnp.float32),
        grid=(B // ni,),
        in_specs=[
            pl.BlockSpec((ni, HW, C), lambda b: (b, 0, 0)),
            pl.BlockSpec((psum, HW), lambda b: (0, 0)),
            pl.BlockSpec((HW, 9 * psum), lambda b: (0, 0)),
            pl.BlockSpec((4, C, C), lambda b: (0, 0, 0)),
            pl.BlockSpec((4, 1, C), lambda b: (0, 0, 0)),
            pl.BlockSpec((4, 1, C), lambda b: (0, 0, 0)),
            pl.BlockSpec((9, C, 5 * C), lambda b: (0, 0, 0)),
            pl.BlockSpec((C, 1), lambda b: (0, 0)),
            pl.BlockSpec((C, 1), lambda b: (0, 0)),
        ],
        out_specs=pl.BlockSpec((ni, C, HW), lambda b: (b, 0, 0)),
        scratch_shapes=[pltpu.VMEM((ni * (H + 2) * W, 3 * C), jnp.bfloat16),
                        pltpu.VMEM((ni * 9 * psum, C), jnp.bfloat16)],
        compiler_params=pltpu.CompilerParams(
            dimension_semantics=("parallel",),
            vmem_limit_bytes=48 * 1024 * 1024,
        ),
        cost_estimate=pl.CostEstimate(flops=flops, transcendentals=0,
                                      bytes_accessed=bytes_accessed),
    )(x_cl, a_stack, l_op, w1, s1, b1, wt, so, bo)

    return out_cf.reshape(B, C, H, W)
